# baseline (speedup 1.0000x reference)
import jax
import jax.numpy as jnp
from jax import lax
from jax.experimental import pallas as pl
from jax.experimental.pallas import tpu as pltpu
from jax.experimental.pallas import tpu_sc as plsc

_NUM_CLASSES = 21
_C = 20
_CONF = 0.01
_IOU = 0.45
_STEPS = 100
_N_RAW = 20000
_N = 20480
_NW = 16
_P = _N // _NW
_NCH = _P // 16
_BIGF = 1e9
_NEGF = -3e38


def _lane():
    return lax.broadcasted_iota(jnp.int32, (16,), 0)


def _bcast_f(x):
    return jnp.full((16,), x, dtype=jnp.float32)


def _bcast_i(x):
    return jnp.full((16,), x, dtype=jnp.int32)


def _nms_body(lg_hbm, bb_hbm, an_hbm, out_hbm, buf_v, sc_v, geom_v, sca_v,
              cache_v, stage_v, exch_v, initstg_v, init_v, out_v, shared_step,
              shared_init, sem):
    wid = lax.axis_index("s")
    cid = lax.axis_index("c")
    base = wid * _P
    lane = _lane()

    srcs = ([lg_hbm.at[pl.ds(r * _N + base, _P)] for r in range(21)]
            + [bb_hbm.at[pl.ds(r * _N + base, _P)] for r in range(4)]
            + [an_hbm.at[pl.ds(r * _N + base, _P)] for r in range(4)])
    cps = [pltpu.async_copy(src, buf_v.at[pl.ds(r * _P, _P)], sem)
           for r, src in enumerate(srcs)]
    for cp in cps:
        cp.wait()

    def prep_chunk(i):
        off = i * 16
        logits = [buf_v[pl.ds(c * _P + off, 16)] for c in range(_NUM_CLASSES)]
        mx = logits[0]
        for c in range(1, _NUM_CLASSES):
            mx = jnp.maximum(mx, logits[c])
        es = [jnp.exp(l - mx) for l in logits]
        ssum = es[0]
        for c in range(1, _NUM_CLASSES):
            ssum = ssum + es[c]
        gidx = base + off + lane
        inb = gidx < _N_RAW
        for c in range(1, _NUM_CLASSES):
            s = es[c] / ssum
            s = jnp.where(inb & (s > _CONF), s, 0.0)
            sc_v[pl.ds((c - 1) * _P + off, 16)] = s
        lcx = buf_v[pl.ds(21 * _P + off, 16)]
        lcy = buf_v[pl.ds(22 * _P + off, 16)]
        lw = buf_v[pl.ds(23 * _P + off, 16)]
        lh = buf_v[pl.ds(24 * _P + off, 16)]
        acx = buf_v[pl.ds(25 * _P + off, 16)]
        acy = buf_v[pl.ds(26 * _P + off, 16)]
        aw = buf_v[pl.ds(27 * _P + off, 16)]
        ah = buf_v[pl.ds(28 * _P + off, 16)]
        cx = lcx * 0.1 * aw + acx
        cy = lcy * 0.1 * ah + acy
        sw = jnp.exp(lw * 0.2) * aw
        sh = jnp.exp(lh * 0.2) * ah
        x1 = cx - sw / 2.0
        y1 = cy - sh / 2.0
        x2 = cx + sw / 2.0
        y2 = cy + sh / 2.0
        geom_v[pl.ds(0 * _P + off, 16)] = x1
        geom_v[pl.ds(1 * _P + off, 16)] = y1
        geom_v[pl.ds(2 * _P + off, 16)] = x2
        geom_v[pl.ds(3 * _P + off, 16)] = y2
        for c in range(_C):
            offs = (c + 1) * 4.0
            a2 = (jnp.maximum((x2 + offs) - (x1 + offs), 0.0)
                  * jnp.maximum((y2 + offs) - (y1 + offs), 0.0))
            sca_v[pl.ds(c * _P + off, 16)] = a2

    plsc.parallel_loop(0, _NCH, unroll=2)(prep_chunk)

    glane = base + lane

    def rescan(cstar, ox1w, oy1w, ox2w, oy2w, warea, offs):
        coff = cstar * _P

        def chunk(i, carry):
            vmax, vj = carry
            off = i * 16
            s = sc_v[pl.ds(coff + off, 16)]
            ox1 = geom_v[pl.ds(0 * _P + off, 16)] + offs
            oy1 = geom_v[pl.ds(1 * _P + off, 16)] + offs
            ox2 = geom_v[pl.ds(2 * _P + off, 16)] + offs
            oy2 = geom_v[pl.ds(3 * _P + off, 16)] + offs
            ltx = jnp.maximum(ox1w, ox1)
            lty = jnp.maximum(oy1w, oy1)
            rbx = jnp.minimum(ox2w, ox2)
            rby = jnp.minimum(oy2w, oy2)
            inter = (jnp.maximum(rbx - ltx, 0.0)
                     * jnp.maximum(rby - lty, 0.0))
            a2 = sca_v[pl.ds(coff + off, 16)]
            iou = inter / jnp.maximum(warea + a2 - inter, 1e-9)
            s = jnp.where(iou > _IOU, 0.0, s)
            sc_v[pl.ds(coff + off, 16)] = s
            upd = s > vmax
            vmax = jnp.where(upd, s, vmax)
            vj = jnp.where(upd, glane + off, vj)
            return vmax, vj

        vmax, vj = plsc.parallel_loop(
            0, _NCH, unroll=4,
            carry=(_bcast_f(-1.0), _bcast_i(0)))(chunk)
        m_loc = jnp.max(vmax)
        j_loc = jnp.min(jnp.where(vmax == m_loc, vj, 2**30))
        ll = _bcast_i(j_loc - base)
        lx1 = plsc.load_gather(geom_v, [ll])
        ly1 = plsc.load_gather(geom_v, [ll + _P])
        lx2 = plsc.load_gather(geom_v, [ll + 2 * _P])
        ly2 = plsc.load_gather(geom_v, [ll + 3 * _P])
        return m_loc, j_loc, lx1, ly1, lx2, ly2

    def assemble(m_loc, j_loc, lx1, ly1, lx2, ly2):
        st = jnp.where(lane == 0, m_loc,
             jnp.where(lane == 1, j_loc.astype(jnp.float32),
             jnp.where(lane == 2, lx1,
             jnp.where(lane == 3, ly1,
             jnp.where(lane == 4, lx2,
             jnp.where(lane == 5, ly2, 0.0))))))
        return st

    def _take(v, idxv):
        return lax.gather(
            v, idxv[:, None],
            dimension_numbers=lax.GatherDimensionNumbers(
                offset_dims=(), collapsed_slice_dims=(0,),
                start_index_map=(0,)),
            slice_sizes=(1,),
            mode=lax.GatherScatterMode.PROMISE_IN_BOUNDS)

    _zerov = _bcast_i(0)

    def combine_into_cache(src_ref, srow, coff, cstar):
        mvec = plsc.load_gather(src_ref, [lane * srow + coff])
        jvec = plsc.load_gather(src_ref, [lane * srow + (coff + 1)])
        gm_c = _bcast_f(jnp.max(mvec))
        jm = jnp.where(mvec == gm_c, jvec, _BIGF)
        jsel = _bcast_f(jnp.min(jm))
        wl = _bcast_i(jnp.min(jnp.where(jm == jsel, lane, 15)))
        vals = [gm_c, jsel]
        for f in range(2, 6):
            cv = plsc.load_gather(src_ref, [lane * srow + (coff + f)])
            vals.append(_take(cv, wl))
        for f in range(6):
            plsc.store_scatter(cache_v, [_bcast_i(f * 32 + cstar)],
                               vals[f], mask=lane == 0)
        return vals

    def cache_candidate(excl):
        m0 = cache_v[pl.ds(0, 16)]
        m1 = cache_v[pl.ds(16, 16)]
        m0 = jnp.where(lane == excl, -1.0, m0)
        m1 = jnp.where(lane + 16 == excl, -1.0, m1)
        j0 = cache_v[pl.ds(32, 16)]
        j1 = cache_v[pl.ds(48, 16)]
        gm = _bcast_f(jnp.maximum(jnp.max(m0), jnp.max(m1)))
        lf = lane.astype(jnp.float32)
        r0 = jnp.where(m0 == gm, j0 * 20.0 + lf, _BIGF)
        r1 = jnp.where(m1 == gm, j1 * 20.0 + (lf + 16.0), _BIGF)
        r = _bcast_f(jnp.minimum(jnp.min(r0), jnp.min(r1)))
        c = lax.rem(r.astype(jnp.int32), _C)
        gidxv = jnp.where(lane < 4, (lane + 2) * 32 + c, c)
        g = plsc.load_gather(cache_v, [gidxv])
        x1 = _take(g, _zerov)
        y1 = _take(g, _zerov + 1)
        x2 = _take(g, _zerov + 2)
        y2 = _take(g, _zerov + 3)
        return gm, r, x1, y1, x2, y2

    cache_v[pl.ds(0, 16)] = _bcast_f(-1.0)
    cache_v[pl.ds(16, 16)] = jnp.where(lane < 4, 0.0, -1.0)

    def init_class(c, carry):
        coff = c * _P

        def chunk(i, carry2):
            vmax, vj = carry2
            off = i * 16
            s = sc_v[pl.ds(coff + off, 16)]
            upd = s > vmax
            return jnp.where(upd, s, vmax), jnp.where(upd, glane + off, vj)

        vmax, vj = plsc.parallel_loop(
            0, _NCH, unroll=8,
            carry=(_bcast_f(-1.0), _bcast_i(0)))(chunk)
        m_loc = jnp.max(vmax)
        j_loc = jnp.min(jnp.where(vmax == m_loc, vj, 2**30))
        ll = _bcast_i(j_loc - base)
        lx1 = jnp.max(plsc.load_gather(geom_v, [ll]))
        ly1 = jnp.max(plsc.load_gather(geom_v, [ll + _P]))
        lx2 = jnp.max(plsc.load_gather(geom_v, [ll + 2 * _P]))
        ly2 = jnp.max(plsc.load_gather(geom_v, [ll + 3 * _P]))
        initstg_v[pl.ds(c * 16, 16)] = assemble(
            m_loc, j_loc, lx1, ly1, lx2, ly2)
        return carry

    lax.fori_loop(0, _C, init_class, 0)
    pltpu.sync_copy(initstg_v, shared_init.at[pl.ds(wid * (_C * 16), _C * 16)])
    plsc.subcore_barrier()
    pltpu.sync_copy(shared_init, init_v)

    def init_reduce(c, carry):
        combine_into_cache(init_v, _C * 16, c * 16, c)
        return carry

    lax.fori_loop(0, _C, init_reduce, 0)

    def step(t, carry):
        gm, r, x1w, y1w, x2w, y2w = carry
        riv = r.astype(jnp.int32)
        cstar = lax.rem(jnp.max(riv), _C)
        cstarv = lax.rem(riv, _C)
        jstarv = lax.div(riv, _C)
        valid = gm > 0.0
        labf = (cstarv + 1).astype(jnp.float32)
        ov = jnp.where(lane == 0, x1w,
             jnp.where(lane == 1, y1w,
             jnp.where(lane == 2, x2w,
             jnp.where(lane == 3, y2w,
             jnp.where(lane == 4, gm,
             jnp.where(lane == 5, labf, 0.0))))))
        dflt = jnp.where(lane == 5, -1.0, 0.0)
        out_v[pl.ds(t * 16, 16)] = jnp.where(valid, ov, dflt)
        offs = labf * 4.0
        ox1w = x1w + offs
        oy1w = y1w + offs
        ox2w = x2w + offs
        oy2w = y2w + offs
        warea = jnp.maximum(ox2w - ox1w, 0.0) * jnp.maximum(oy2w - oy1w, 0.0)
        jl = jstarv - base
        own = (jl >= 0) & (jl < _P)
        jl = jnp.clip(jl, 0, _P - 1)
        plsc.store_scatter(sc_v, [cstarv * _P + jl],
                           _bcast_f(0.0), mask=(lane == 0) & own)
        m_loc, j_loc, lx1, ly1, lx2, ly2 = rescan(
            cstar, ox1w, oy1w, ox2w, oy2w, warea, offs)
        stage_v[pl.ds(0, 16)] = assemble(m_loc, j_loc, lx1, ly1, lx2, ly2)
        boff = lax.rem(t, 2) * 128
        pltpu.sync_copy(stage_v.at[pl.ds(0, 8)],
                        shared_step.at[pl.ds(boff + wid * 8, 8)])
        plsc.subcore_barrier()
        rd = pltpu.async_copy(shared_step.at[pl.ds(boff, 128)], exch_v, sem)
        gm_r, r_r, rx1, ry1, rx2, ry2 = cache_candidate(cstar)
        rd.wait()
        vals = combine_into_cache(exch_v, 8, 0, cstar)
        m_new, j_new, nx1, ny1, nx2, ny2 = vals
        rank_new = j_new * 20.0 + cstarv.astype(jnp.float32)
        take = (m_new > gm_r) | ((m_new == gm_r) & (rank_new < r_r))
        return (jnp.where(take, m_new, gm_r),
                jnp.where(take, rank_new, r_r),
                jnp.where(take, nx1, rx1),
                jnp.where(take, ny1, ry1),
                jnp.where(take, nx2, rx2),
                jnp.where(take, ny2, ry2))

    lax.fori_loop(0, _STEPS, step, cache_candidate(jnp.int32(-1)))

    @pl.when(jnp.logical_and(cid == 0, wid == 0))
    def _():
        pltpu.sync_copy(out_v, out_hbm)


@jax.jit
def _run(lg, bb, an):
    mesh = plsc.VectorSubcoreMesh(core_axis_name="c", subcore_axis_name="s",
                                  num_cores=2, num_subcores=_NW)
    f = pl.kernel(
        _nms_body,
        mesh=mesh,
        compiler_params=pltpu.CompilerParams(needs_layout_passes=False),
        out_type=jax.ShapeDtypeStruct((_STEPS * 16,), jnp.float32),
        scratch_types=[
            pltpu.VMEM((29 * _P,), jnp.float32),
            pltpu.VMEM((_C * _P,), jnp.float32),
            pltpu.VMEM((4 * _P,), jnp.float32),
            pltpu.VMEM((_C * _P,), jnp.float32),
            pltpu.VMEM((256,), jnp.float32),
            pltpu.VMEM((16,), jnp.float32),
            pltpu.VMEM((128,), jnp.float32),
            pltpu.VMEM((_C * 16,), jnp.float32),
            pltpu.VMEM((_NW * _C * 16,), jnp.float32),
            pltpu.VMEM((_STEPS * 16,), jnp.float32),
            pltpu.VMEM_SHARED((2 * _NW * 8,), jnp.float32),
            pltpu.VMEM_SHARED((_NW * _C * 16,), jnp.float32),
            pltpu.SemaphoreType.DMA,
        ],
    )
    return f(lg, bb, an)


def kernel(cls_logits, bbox_pred, anchors):
    pad = _N - _N_RAW
    lg = jnp.pad(cls_logits[0].T, ((0, 0), (0, pad))).reshape(-1)
    bb = jnp.pad(bbox_pred[0].T, ((0, 0), (0, pad))).reshape(-1)
    an = jnp.pad(anchors.T, ((0, 0), (0, pad))).reshape(-1)
    out = _run(lg, bb, an).reshape(_STEPS, 16)
    kb = out[:, 0:4]
    ks = out[:, 4]
    kl = out[:, 5].astype(jnp.int32)
    return kb, ks, kl


# speedup vs baseline: 1.0425x; 1.0425x over previous
import jax
import jax.numpy as jnp
from jax import lax
from jax.experimental import pallas as pl
from jax.experimental.pallas import tpu as pltpu
from jax.experimental.pallas import tpu_sc as plsc

_NUM_CLASSES = 21
_C = 20
_CONF = 0.01
_IOU = 0.45
_STEPS = 100
_N_RAW = 20000
_N = 20480
_NW = 16
_P = _N // _NW
_NCH = _P // 16
_BIGF = 1e9
_NEGF = -3e38


def _lane():
    return lax.broadcasted_iota(jnp.int32, (16,), 0)


def _bcast_f(x):
    return jnp.full((16,), x, dtype=jnp.float32)


def _bcast_i(x):
    return jnp.full((16,), x, dtype=jnp.int32)


def _nms_body(lg_hbm, bb_hbm, an_hbm, out_hbm, buf_v, sc_v, geom_v, sca_v,
              cache_v, stage_v, exch_v, initstg_v, init_v, out_v, shared_step,
              shared_init, sem):
    wid = lax.axis_index("s")
    cid = lax.axis_index("c")
    base = wid * _P
    lane = _lane()

    srcs = ([lg_hbm.at[pl.ds(r * _N + base, _P)] for r in range(21)]
            + [bb_hbm.at[pl.ds(r * _N + base, _P)] for r in range(4)]
            + [an_hbm.at[pl.ds(r * _N + base, _P)] for r in range(4)])
    cps = [pltpu.async_copy(src, buf_v.at[pl.ds(r * _P, _P)], sem)
           for r, src in enumerate(srcs)]
    for cp in cps:
        cp.wait()

    def prep_chunk(i):
        off = i * 16
        logits = [buf_v[pl.ds(c * _P + off, 16)] for c in range(_NUM_CLASSES)]
        mx = logits[0]
        for c in range(1, _NUM_CLASSES):
            mx = jnp.maximum(mx, logits[c])
        es = [jnp.exp(l - mx) for l in logits]
        ssum = es[0]
        for c in range(1, _NUM_CLASSES):
            ssum = ssum + es[c]
        gidx = base + off + lane
        inb = gidx < _N_RAW
        for c in range(1, _NUM_CLASSES):
            s = es[c] / ssum
            s = jnp.where(inb & (s > _CONF), s, 0.0)
            sc_v[pl.ds((c - 1) * _P + off, 16)] = s
        lcx = buf_v[pl.ds(21 * _P + off, 16)]
        lcy = buf_v[pl.ds(22 * _P + off, 16)]
        lw = buf_v[pl.ds(23 * _P + off, 16)]
        lh = buf_v[pl.ds(24 * _P + off, 16)]
        acx = buf_v[pl.ds(25 * _P + off, 16)]
        acy = buf_v[pl.ds(26 * _P + off, 16)]
        aw = buf_v[pl.ds(27 * _P + off, 16)]
        ah = buf_v[pl.ds(28 * _P + off, 16)]
        cx = lcx * 0.1 * aw + acx
        cy = lcy * 0.1 * ah + acy
        sw = jnp.exp(lw * 0.2) * aw
        sh = jnp.exp(lh * 0.2) * ah
        x1 = cx - sw / 2.0
        y1 = cy - sh / 2.0
        x2 = cx + sw / 2.0
        y2 = cy + sh / 2.0
        geom_v[pl.ds(0 * _P + off, 16)] = x1
        geom_v[pl.ds(1 * _P + off, 16)] = y1
        geom_v[pl.ds(2 * _P + off, 16)] = x2
        geom_v[pl.ds(3 * _P + off, 16)] = y2
        for c in range(_C):
            offs = (c + 1) * 4.0
            a2 = (jnp.maximum((x2 + offs) - (x1 + offs), 0.0)
                  * jnp.maximum((y2 + offs) - (y1 + offs), 0.0))
            sca_v[pl.ds(c * _P + off, 16)] = a2

    plsc.parallel_loop(0, _NCH, unroll=2)(prep_chunk)

    glane = base + lane

    def rescan(cstar, ox1w, oy1w, ox2w, oy2w, warea, offs):
        coff = cstar * _P

        def chunk(i, carry):
            vmax, vj = carry
            off = i * 16
            s = sc_v[pl.ds(coff + off, 16)]
            ox1 = geom_v[pl.ds(0 * _P + off, 16)] + offs
            oy1 = geom_v[pl.ds(1 * _P + off, 16)] + offs
            ox2 = geom_v[pl.ds(2 * _P + off, 16)] + offs
            oy2 = geom_v[pl.ds(3 * _P + off, 16)] + offs
            ltx = jnp.maximum(ox1w, ox1)
            lty = jnp.maximum(oy1w, oy1)
            rbx = jnp.minimum(ox2w, ox2)
            rby = jnp.minimum(oy2w, oy2)
            inter = (jnp.maximum(rbx - ltx, 0.0)
                     * jnp.maximum(rby - lty, 0.0))
            a2 = sca_v[pl.ds(coff + off, 16)]
            iou = inter / jnp.maximum(warea + a2 - inter, 1e-9)
            s = jnp.where(iou > _IOU, 0.0, s)
            sc_v[pl.ds(coff + off, 16)] = s
            upd = s > vmax
            vmax = jnp.where(upd, s, vmax)
            vj = jnp.where(upd, glane + off, vj)
            return vmax, vj

        vmax, vj = plsc.parallel_loop(
            0, _NCH, unroll=4,
            carry=(_bcast_f(-1.0), _bcast_i(0)))(chunk)
        m_loc = jnp.max(vmax)
        j_loc = jnp.min(jnp.where(vmax == m_loc, vj, 2**30))
        ll = _bcast_i(j_loc - base)
        lx1 = jnp.max(plsc.load_gather(geom_v, [ll]))
        ly1 = jnp.max(plsc.load_gather(geom_v, [ll + _P]))
        lx2 = jnp.max(plsc.load_gather(geom_v, [ll + 2 * _P]))
        ly2 = jnp.max(plsc.load_gather(geom_v, [ll + 3 * _P]))
        return m_loc, j_loc, lx1, ly1, lx2, ly2

    def assemble(m_loc, j_loc, lx1, ly1, lx2, ly2):
        st = jnp.where(lane == 0, m_loc,
             jnp.where(lane == 1, j_loc.astype(jnp.float32),
             jnp.where(lane == 2, lx1,
             jnp.where(lane == 3, ly1,
             jnp.where(lane == 4, lx2,
             jnp.where(lane == 5, ly2, 0.0))))))
        return st

    def combine_into_cache(src_ref, srow, coff, cstar):
        mvec = plsc.load_gather(src_ref, [lane * srow + coff])
        jvec = plsc.load_gather(src_ref, [lane * srow + (coff + 1)])
        gm_c = jnp.max(mvec)
        jm = jnp.where(mvec == gm_c, jvec, _BIGF)
        jsel = jnp.min(jm)
        selm = jm == jsel
        vals = [gm_c, jsel]
        for f in range(2, 6):
            cv = plsc.load_gather(src_ref, [lane * srow + (coff + f)])
            vals.append(jnp.max(jnp.where(selm, cv, _NEGF)))
        for f in range(6):
            plsc.store_scatter(cache_v, [_bcast_i(f * 32 + cstar)],
                               _bcast_f(vals[f]), mask=lane == 0)
        return vals

    def cache_candidate(excl):
        m0 = cache_v[pl.ds(0, 16)]
        m1 = cache_v[pl.ds(16, 16)]
        m0 = jnp.where(lane == excl, -1.0, m0)
        m1 = jnp.where(lane + 16 == excl, -1.0, m1)
        j0 = cache_v[pl.ds(32, 16)]
        j1 = cache_v[pl.ds(48, 16)]
        gm = jnp.maximum(jnp.max(m0), jnp.max(m1))
        lf = lane.astype(jnp.float32)
        r0 = jnp.where(m0 == gm, j0 * 20.0 + lf, _BIGF)
        r1 = jnp.where(m1 == gm, j1 * 20.0 + (lf + 16.0), _BIGF)
        r = jnp.minimum(jnp.min(r0), jnp.min(r1))
        c = lax.rem(r.astype(jnp.int32), _C)
        gidxv = jnp.where(lane < 4, (lane + 2) * 32 + c, c)
        g = plsc.load_gather(cache_v, [gidxv])
        x1 = jnp.max(jnp.where(lane == 0, g, _NEGF))
        y1 = jnp.max(jnp.where(lane == 1, g, _NEGF))
        x2 = jnp.max(jnp.where(lane == 2, g, _NEGF))
        y2 = jnp.max(jnp.where(lane == 3, g, _NEGF))
        return gm, r, x1, y1, x2, y2

    cache_v[pl.ds(0, 16)] = _bcast_f(-1.0)
    cache_v[pl.ds(16, 16)] = jnp.where(lane < 4, 0.0, -1.0)

    def init_class(c, carry):
        coff = c * _P

        def chunk(i, carry2):
            vmax, vj = carry2
            off = i * 16
            s = sc_v[pl.ds(coff + off, 16)]
            upd = s > vmax
            return jnp.where(upd, s, vmax), jnp.where(upd, glane + off, vj)

        vmax, vj = plsc.parallel_loop(
            0, _NCH, unroll=8,
            carry=(_bcast_f(-1.0), _bcast_i(0)))(chunk)
        m_loc = jnp.max(vmax)
        j_loc = jnp.min(jnp.where(vmax == m_loc, vj, 2**30))
        ll = _bcast_i(j_loc - base)
        lx1 = jnp.max(plsc.load_gather(geom_v, [ll]))
        ly1 = jnp.max(plsc.load_gather(geom_v, [ll + _P]))
        lx2 = jnp.max(plsc.load_gather(geom_v, [ll + 2 * _P]))
        ly2 = jnp.max(plsc.load_gather(geom_v, [ll + 3 * _P]))
        initstg_v[pl.ds(c * 16, 16)] = assemble(
            m_loc, j_loc, lx1, ly1, lx2, ly2)
        return carry

    lax.fori_loop(0, _C, init_class, 0)
    pltpu.sync_copy(initstg_v, shared_init.at[pl.ds(wid * (_C * 16), _C * 16)])
    plsc.subcore_barrier()
    pltpu.sync_copy(shared_init, init_v)

    def init_reduce(c, carry):
        combine_into_cache(init_v, _C * 16, c * 16, c)
        return carry

    lax.fori_loop(0, _C, init_reduce, 0)

    def step(t, carry):
        gm, r, x1w, y1w, x2w, y2w = carry
        ri = r.astype(jnp.int32)
        cstar = lax.rem(ri, _C)
        jstar = lax.div(ri, _C)
        valid = gm > 0.0
        labf = (cstar + 1).astype(jnp.float32)
        ov = jnp.where(lane == 0, x1w,
             jnp.where(lane == 1, y1w,
             jnp.where(lane == 2, x2w,
             jnp.where(lane == 3, y2w,
             jnp.where(lane == 4, gm,
             jnp.where(lane == 5, labf, 0.0))))))
        dflt = jnp.where(lane == 5, -1.0, 0.0)
        out_v[pl.ds(t * 16, 16)] = jnp.where(valid, ov, dflt)
        offs = labf * 4.0
        ox1w = x1w + offs
        oy1w = y1w + offs
        ox2w = x2w + offs
        oy2w = y2w + offs
        warea = jnp.maximum(ox2w - ox1w, 0.0) * jnp.maximum(oy2w - oy1w, 0.0)
        jl = jstar - base
        own = (jl >= 0) & (jl < _P)
        jl = jnp.clip(jl, 0, _P - 1)
        plsc.store_scatter(sc_v, [_bcast_i(cstar * _P + jl)],
                           _bcast_f(0.0), mask=(lane == 0) & own)
        m_loc, j_loc, lx1, ly1, lx2, ly2 = rescan(
            cstar, ox1w, oy1w, ox2w, oy2w, warea, offs)
        stage_v[pl.ds(0, 16)] = assemble(m_loc, j_loc, lx1, ly1, lx2, ly2)
        boff = lax.rem(t, 2) * 128
        pltpu.sync_copy(stage_v.at[pl.ds(0, 8)],
                        shared_step.at[pl.ds(boff + wid * 8, 8)])
        plsc.subcore_barrier()
        rd = pltpu.async_copy(shared_step.at[pl.ds(boff, 128)], exch_v, sem)
        gm_r, r_r, rx1, ry1, rx2, ry2 = cache_candidate(cstar)
        rd.wait()
        vals = combine_into_cache(exch_v, 8, 0, cstar)
        m_new, j_new, nx1, ny1, nx2, ny2 = vals
        rank_new = j_new * 20.0 + cstar.astype(jnp.float32)
        take = (m_new > gm_r) | ((m_new == gm_r) & (rank_new < r_r))
        return (jnp.where(take, m_new, gm_r),
                jnp.where(take, rank_new, r_r),
                jnp.where(take, nx1, rx1),
                jnp.where(take, ny1, ry1),
                jnp.where(take, nx2, rx2),
                jnp.where(take, ny2, ry2))

    lax.fori_loop(0, _STEPS, step, cache_candidate(jnp.int32(-1)))

    @pl.when(jnp.logical_and(cid == 0, wid == 0))
    def _():
        pltpu.sync_copy(out_v, out_hbm)


@jax.jit
def _run(lg, bb, an):
    mesh = plsc.VectorSubcoreMesh(core_axis_name="c", subcore_axis_name="s",
                                  num_cores=2, num_subcores=_NW)
    f = pl.kernel(
        _nms_body,
        mesh=mesh,
        compiler_params=pltpu.CompilerParams(needs_layout_passes=False),
        out_type=jax.ShapeDtypeStruct((_STEPS * 16,), jnp.float32),
        scratch_types=[
            pltpu.VMEM((29 * _P,), jnp.float32),
            pltpu.VMEM((_C * _P,), jnp.float32),
            pltpu.VMEM((4 * _P,), jnp.float32),
            pltpu.VMEM((_C * _P,), jnp.float32),
            pltpu.VMEM((256,), jnp.float32),
            pltpu.VMEM((16,), jnp.float32),
            pltpu.VMEM((128,), jnp.float32),
            pltpu.VMEM((_C * 16,), jnp.float32),
            pltpu.VMEM((_NW * _C * 16,), jnp.float32),
            pltpu.VMEM((_STEPS * 16,), jnp.float32),
            pltpu.VMEM_SHARED((2 * _NW * 8,), jnp.float32),
            pltpu.VMEM_SHARED((_NW * _C * 16,), jnp.float32),
            pltpu.SemaphoreType.DMA,
        ],
    )
    return f(lg, bb, an)


def kernel(cls_logits, bbox_pred, anchors):
    pad = _N - _N_RAW
    lg = jnp.pad(cls_logits[0].T, ((0, 0), (0, pad))).reshape(-1)
    bb = jnp.pad(bbox_pred[0].T, ((0, 0), (0, pad))).reshape(-1)
    an = jnp.pad(anchors.T, ((0, 0), (0, pad))).reshape(-1)
    out = _run(lg, bb, an).reshape(_STEPS, 16)
    kb = out[:, 0:4]
    ks = out[:, 4]
    kl = out[:, 5].astype(jnp.int32)
    return kb, ks, kl


# speedup vs baseline: 1.0733x; 1.0296x over previous
import jax
import jax.numpy as jnp
from jax import lax
from jax.experimental import pallas as pl
from jax.experimental.pallas import tpu as pltpu
from jax.experimental.pallas import tpu_sc as plsc

_NUM_CLASSES = 21
_C = 20
_CONF = 0.01
_IOU = 0.45
_STEPS = 100
_N_RAW = 20000
_N = 20480
_NW = 16
_P = _N // _NW
_NCH = _P // 16
_BIGF = 1e9
_NEGF = -3e38


def _lane():
    return lax.broadcasted_iota(jnp.int32, (16,), 0)


def _bcast_f(x):
    return jnp.full((16,), x, dtype=jnp.float32)


def _bcast_i(x):
    return jnp.full((16,), x, dtype=jnp.int32)


def _nms_body(lg_hbm, bb_hbm, an_hbm, out_hbm, buf_v, sc_v, geom_v, sca_v,
              cache_v, stage_v, exch_v, initstg_v, init_v, out_v, shared_step,
              shared_init, sem):
    wid = lax.axis_index("s")
    cid = lax.axis_index("c")
    base = wid * _P
    lane = _lane()

    srcs = ([lg_hbm.at[pl.ds(r * _N + base, _P)] for r in range(21)]
            + [bb_hbm.at[pl.ds(r * _N + base, _P)] for r in range(4)]
            + [an_hbm.at[pl.ds(r * _N + base, _P)] for r in range(4)])
    cps = [pltpu.async_copy(src, buf_v.at[pl.ds(r * _P, _P)], sem)
           for r, src in enumerate(srcs)]
    for cp in cps:
        cp.wait()

    def prep_chunk(i):
        off = i * 16
        logits = [buf_v[pl.ds(c * _P + off, 16)] for c in range(_NUM_CLASSES)]
        mx = logits[0]
        for c in range(1, _NUM_CLASSES):
            mx = jnp.maximum(mx, logits[c])
        es = [jnp.exp(l - mx) for l in logits]
        ssum = es[0]
        for c in range(1, _NUM_CLASSES):
            ssum = ssum + es[c]
        gidx = base + off + lane
        inb = gidx < _N_RAW
        for c in range(1, _NUM_CLASSES):
            s = es[c] / ssum
            s = jnp.where(inb & (s > _CONF), s, 0.0)
            sc_v[pl.ds((c - 1) * _P + off, 16)] = s
        lcx = buf_v[pl.ds(21 * _P + off, 16)]
        lcy = buf_v[pl.ds(22 * _P + off, 16)]
        lw = buf_v[pl.ds(23 * _P + off, 16)]
        lh = buf_v[pl.ds(24 * _P + off, 16)]
        acx = buf_v[pl.ds(25 * _P + off, 16)]
        acy = buf_v[pl.ds(26 * _P + off, 16)]
        aw = buf_v[pl.ds(27 * _P + off, 16)]
        ah = buf_v[pl.ds(28 * _P + off, 16)]
        cx = lcx * 0.1 * aw + acx
        cy = lcy * 0.1 * ah + acy
        sw = jnp.exp(lw * 0.2) * aw
        sh = jnp.exp(lh * 0.2) * ah
        x1 = cx - sw / 2.0
        y1 = cy - sh / 2.0
        x2 = cx + sw / 2.0
        y2 = cy + sh / 2.0
        geom_v[pl.ds(0 * _P + off, 16)] = x1
        geom_v[pl.ds(1 * _P + off, 16)] = y1
        geom_v[pl.ds(2 * _P + off, 16)] = x2
        geom_v[pl.ds(3 * _P + off, 16)] = y2
        for c in range(_C):
            offs = (c + 1) * 4.0
            a2 = (jnp.maximum((x2 + offs) - (x1 + offs), 0.0)
                  * jnp.maximum((y2 + offs) - (y1 + offs), 0.0))
            sca_v[pl.ds(c * _P + off, 16)] = a2

    plsc.parallel_loop(0, _NCH, unroll=4)(prep_chunk)

    glane = base + lane

    def rescan(cstar, ox1w, oy1w, ox2w, oy2w, warea, offs):
        coff = cstar * _P

        def chunk(i, carry):
            vmax, vj = carry
            off = i * 16
            s = sc_v[pl.ds(coff + off, 16)]
            ox1 = geom_v[pl.ds(0 * _P + off, 16)] + offs
            oy1 = geom_v[pl.ds(1 * _P + off, 16)] + offs
            ox2 = geom_v[pl.ds(2 * _P + off, 16)] + offs
            oy2 = geom_v[pl.ds(3 * _P + off, 16)] + offs
            ltx = jnp.maximum(ox1w, ox1)
            lty = jnp.maximum(oy1w, oy1)
            rbx = jnp.minimum(ox2w, ox2)
            rby = jnp.minimum(oy2w, oy2)
            inter = (jnp.maximum(rbx - ltx, 0.0)
                     * jnp.maximum(rby - lty, 0.0))
            a2 = sca_v[pl.ds(coff + off, 16)]
            iou = inter / jnp.maximum(warea + a2 - inter, 1e-9)
            s = jnp.where(iou > _IOU, 0.0, s)
            sc_v[pl.ds(coff + off, 16)] = s
            upd = s > vmax
            vmax = jnp.where(upd, s, vmax)
            vj = jnp.where(upd, glane + off, vj)
            return vmax, vj

        vmax, vj = plsc.parallel_loop(
            0, _NCH, unroll=4,
            carry=(_bcast_f(-1.0), _bcast_i(0)))(chunk)
        m_loc = jnp.max(vmax)
        j_loc = jnp.min(jnp.where(vmax == m_loc, vj, 2**30))
        ll = _bcast_i(j_loc - base)
        lx1 = jnp.max(plsc.load_gather(geom_v, [ll]))
        ly1 = jnp.max(plsc.load_gather(geom_v, [ll + _P]))
        lx2 = jnp.max(plsc.load_gather(geom_v, [ll + 2 * _P]))
        ly2 = jnp.max(plsc.load_gather(geom_v, [ll + 3 * _P]))
        return m_loc, j_loc, lx1, ly1, lx2, ly2

    def assemble(m_loc, j_loc, lx1, ly1, lx2, ly2):
        st = jnp.where(lane == 0, m_loc,
             jnp.where(lane == 1, j_loc.astype(jnp.float32),
             jnp.where(lane == 2, lx1,
             jnp.where(lane == 3, ly1,
             jnp.where(lane == 4, lx2,
             jnp.where(lane == 5, ly2, 0.0))))))
        return st

    def combine_into_cache(src_ref, srow, coff, cstar):
        mvec = plsc.load_gather(src_ref, [lane * srow + coff])
        jvec = plsc.load_gather(src_ref, [lane * srow + (coff + 1)])
        gm_c = jnp.max(mvec)
        jm = jnp.where(mvec == gm_c, jvec, _BIGF)
        jsel = jnp.min(jm)
        selm = jm == jsel
        vals = [gm_c, jsel]
        for f in range(2, 6):
            cv = plsc.load_gather(src_ref, [lane * srow + (coff + f)])
            vals.append(jnp.max(jnp.where(selm, cv, _NEGF)))
        for f in range(6):
            plsc.store_scatter(cache_v, [_bcast_i(f * 32 + cstar)],
                               _bcast_f(vals[f]), mask=lane == 0)
        return vals

    def cache_candidate(excl):
        m0 = cache_v[pl.ds(0, 16)]
        m1 = cache_v[pl.ds(16, 16)]
        m0 = jnp.where(lane == excl, -1.0, m0)
        m1 = jnp.where(lane + 16 == excl, -1.0, m1)
        j0 = cache_v[pl.ds(32, 16)]
        j1 = cache_v[pl.ds(48, 16)]
        gm = jnp.maximum(jnp.max(m0), jnp.max(m1))
        lf = lane.astype(jnp.float32)
        r0 = jnp.where(m0 == gm, j0 * 20.0 + lf, _BIGF)
        r1 = jnp.where(m1 == gm, j1 * 20.0 + (lf + 16.0), _BIGF)
        r = jnp.minimum(jnp.min(r0), jnp.min(r1))
        c = lax.rem(r.astype(jnp.int32), _C)
        gidxv = jnp.where(lane < 4, (lane + 2) * 32 + c, c)
        g = plsc.load_gather(cache_v, [gidxv])
        x1 = jnp.max(jnp.where(lane == 0, g, _NEGF))
        y1 = jnp.max(jnp.where(lane == 1, g, _NEGF))
        x2 = jnp.max(jnp.where(lane == 2, g, _NEGF))
        y2 = jnp.max(jnp.where(lane == 3, g, _NEGF))
        return gm, r, x1, y1, x2, y2

    cache_v[pl.ds(0, 16)] = _bcast_f(-1.0)
    cache_v[pl.ds(16, 16)] = jnp.where(lane < 4, 0.0, -1.0)

    def init_class(c, carry):
        coff = c * _P

        def chunk(i, carry2):
            vmax, vj = carry2
            off = i * 16
            s = sc_v[pl.ds(coff + off, 16)]
            upd = s > vmax
            return jnp.where(upd, s, vmax), jnp.where(upd, glane + off, vj)

        vmax, vj = plsc.parallel_loop(
            0, _NCH, unroll=8,
            carry=(_bcast_f(-1.0), _bcast_i(0)))(chunk)
        m_loc = jnp.max(vmax)
        j_loc = jnp.min(jnp.where(vmax == m_loc, vj, 2**30))
        ll = _bcast_i(j_loc - base)
        lx1 = jnp.max(plsc.load_gather(geom_v, [ll]))
        ly1 = jnp.max(plsc.load_gather(geom_v, [ll + _P]))
        lx2 = jnp.max(plsc.load_gather(geom_v, [ll + 2 * _P]))
        ly2 = jnp.max(plsc.load_gather(geom_v, [ll + 3 * _P]))
        initstg_v[pl.ds(c * 16, 16)] = assemble(
            m_loc, j_loc, lx1, ly1, lx2, ly2)
        return carry

    lax.fori_loop(0, _C, init_class, 0)
    pltpu.sync_copy(initstg_v, shared_init.at[pl.ds(wid * (_C * 16), _C * 16)])
    plsc.subcore_barrier()
    pltpu.sync_copy(shared_init, init_v)

    def init_reduce(c, carry):
        combine_into_cache(init_v, _C * 16, c * 16, c)
        return carry

    lax.fori_loop(0, _C, init_reduce, 0)

    def step(t, carry):
        gm, r, x1w, y1w, x2w, y2w = carry
        ri = r.astype(jnp.int32)
        cstar = lax.rem(ri, _C)
        jstar = lax.div(ri, _C)
        valid = gm > 0.0
        labf = (cstar + 1).astype(jnp.float32)
        ov = jnp.where(lane == 0, x1w,
             jnp.where(lane == 1, y1w,
             jnp.where(lane == 2, x2w,
             jnp.where(lane == 3, y2w,
             jnp.where(lane == 4, gm,
             jnp.where(lane == 5, labf, 0.0))))))
        dflt = jnp.where(lane == 5, -1.0, 0.0)
        out_v[pl.ds(t * 16, 16)] = jnp.where(valid, ov, dflt)
        offs = labf * 4.0
        ox1w = x1w + offs
        oy1w = y1w + offs
        ox2w = x2w + offs
        oy2w = y2w + offs
        warea = jnp.maximum(ox2w - ox1w, 0.0) * jnp.maximum(oy2w - oy1w, 0.0)
        jl = jstar - base
        own = (jl >= 0) & (jl < _P)
        jl = jnp.clip(jl, 0, _P - 1)
        plsc.store_scatter(sc_v, [_bcast_i(cstar * _P + jl)],
                           _bcast_f(0.0), mask=(lane == 0) & own)
        m_loc, j_loc, lx1, ly1, lx2, ly2 = rescan(
            cstar, ox1w, oy1w, ox2w, oy2w, warea, offs)
        stage_v[pl.ds(0, 16)] = assemble(m_loc, j_loc, lx1, ly1, lx2, ly2)
        boff = lax.rem(t, 2) * 128
        pltpu.sync_copy(stage_v.at[pl.ds(0, 8)],
                        shared_step.at[pl.ds(boff + wid * 8, 8)])
        plsc.subcore_barrier()
        rd = pltpu.async_copy(shared_step.at[pl.ds(boff, 128)], exch_v, sem)
        gm_r, r_r, rx1, ry1, rx2, ry2 = cache_candidate(cstar)
        rd.wait()
        vals = combine_into_cache(exch_v, 8, 0, cstar)
        m_new, j_new, nx1, ny1, nx2, ny2 = vals
        rank_new = j_new * 20.0 + cstar.astype(jnp.float32)
        take = (m_new > gm_r) | ((m_new == gm_r) & (rank_new < r_r))
        return (jnp.where(take, m_new, gm_r),
                jnp.where(take, rank_new, r_r),
                jnp.where(take, nx1, rx1),
                jnp.where(take, ny1, ry1),
                jnp.where(take, nx2, rx2),
                jnp.where(take, ny2, ry2))

    lax.fori_loop(0, _STEPS, step, cache_candidate(jnp.int32(-1)))

    @pl.when(jnp.logical_and(cid == 0, wid == 0))
    def _():
        pltpu.sync_copy(out_v, out_hbm)


@jax.jit
def _run(lg, bb, an):
    mesh = plsc.VectorSubcoreMesh(core_axis_name="c", subcore_axis_name="s",
                                  num_cores=2, num_subcores=_NW)
    f = pl.kernel(
        _nms_body,
        mesh=mesh,
        compiler_params=pltpu.CompilerParams(needs_layout_passes=False),
        out_type=jax.ShapeDtypeStruct((_STEPS * 16,), jnp.float32),
        scratch_types=[
            pltpu.VMEM((29 * _P,), jnp.float32),
            pltpu.VMEM((_C * _P,), jnp.float32),
            pltpu.VMEM((4 * _P,), jnp.float32),
            pltpu.VMEM((_C * _P,), jnp.float32),
            pltpu.VMEM((256,), jnp.float32),
            pltpu.VMEM((16,), jnp.float32),
            pltpu.VMEM((128,), jnp.float32),
            pltpu.VMEM((_C * 16,), jnp.float32),
            pltpu.VMEM((_NW * _C * 16,), jnp.float32),
            pltpu.VMEM((_STEPS * 16,), jnp.float32),
            pltpu.VMEM_SHARED((2 * _NW * 8,), jnp.float32),
            pltpu.VMEM_SHARED((_NW * _C * 16,), jnp.float32),
            pltpu.SemaphoreType.DMA,
        ],
    )
    return f(lg, bb, an)


def kernel(cls_logits, bbox_pred, anchors):
    pad = _N - _N_RAW
    lg = jnp.pad(cls_logits[0].T, ((0, 0), (0, pad))).reshape(-1)
    bb = jnp.pad(bbox_pred[0].T, ((0, 0), (0, pad))).reshape(-1)
    an = jnp.pad(anchors.T, ((0, 0), (0, pad))).reshape(-1)
    out = _run(lg, bb, an).reshape(_STEPS, 16)
    kb = out[:, 0:4]
    ks = out[:, 4]
    kl = out[:, 5].astype(jnp.int32)
    return kb, ks, kl
